# Initial kernel scaffold; baseline (speedup 1.0000x reference)
import jax
import jax.numpy as jnp
from jax.experimental import pallas as pl


def kernel(x, edge_index, W1, b1, W2, b2, W3, b3):
    raise NotImplementedError("write your pallas kernel here")


# speedup vs baseline: 9.7343x; 9.7343x over previous
import functools

import jax
import jax.numpy as jnp
from jax import lax
from jax.experimental import pallas as pl
from jax.experimental.pallas import tpu as pltpu
from jax.experimental.pallas import tpu_sc as plsc

N = 10000
N_PAD = 10240
E = 320000
NC = 2
NS = 16
NW = NC * NS
EPT = E // NW
CHUNK = 80
NCHUNK = EPT // CHUNK
ROWS_PT = N_PAD // NS
DEG_W = 16

ROW_BLK = 400


def _sc_mesh():
  return plsc.VectorSubcoreMesh(core_axis_name="c", subcore_axis_name="s")


@functools.partial(
    pl.kernel,
    out_type=jax.ShapeDtypeStruct((NC, N_PAD, DEG_W), jnp.float32),
    mesh=_sc_mesh(),
    scratch_types=[
        pltpu.VMEM((CHUNK,), jnp.int32),
        pltpu.VMEM((CHUNK, DEG_W), jnp.float32),
        pltpu.VMEM_SHARED((N_PAD, DEG_W), jnp.float32),
    ],
    compiler_params=pltpu.CompilerParams(use_tc_tiling_on_sc=False),
)
def _sc_degree(dst_hbm, ones_hbm, zeros_hbm, out_hbm, didx, ones_v, acc):
  c = lax.axis_index("c")
  s = lax.axis_index("s")
  r0 = s * ROWS_PT
  pltpu.sync_copy(ones_hbm, ones_v)
  pltpu.sync_copy(zeros_hbm.at[pl.ds(r0, ROWS_PT)], acc.at[pl.ds(r0, ROWS_PT)])
  plsc.subcore_barrier()
  ebase = (c * NS + s) * EPT

  def body(i, _):
    base = ebase + i * CHUNK
    pltpu.sync_copy(dst_hbm.at[pl.ds(base, CHUNK)], didx)
    pltpu.sync_copy(ones_v, acc.at[didx], add=True)
    return ()

  lax.fori_loop(0, NCHUNK, body, ())
  plsc.subcore_barrier()
  pltpu.sync_copy(acc.at[pl.ds(r0, ROWS_PT)],
                  out_hbm.at[c].at[pl.ds(r0, ROWS_PT)])


def _make_prop(D):
  @functools.partial(
      pl.kernel,
      out_type=jax.ShapeDtypeStruct((NC, N_PAD, D), jnp.float32),
      mesh=_sc_mesh(),
      scratch_types=[
          pltpu.VMEM((CHUNK,), jnp.int32),
          pltpu.VMEM((CHUNK,), jnp.int32),
          pltpu.VMEM((CHUNK, D), jnp.float32),
          pltpu.VMEM_SHARED((N_PAD, D), jnp.float32),
          pltpu.SemaphoreType.DMA,
      ],
      compiler_params=pltpu.CompilerParams(use_tc_tiling_on_sc=False),
  )
  def prop(table_hbm, src_hbm, dst_hbm, zeros_hbm, out_hbm,
           sidx, didx, rows, acc, sem):
    c = lax.axis_index("c")
    s = lax.axis_index("s")
    r0 = s * ROWS_PT
    pltpu.sync_copy(zeros_hbm.at[pl.ds(r0, ROWS_PT)],
                    acc.at[pl.ds(r0, ROWS_PT)])
    plsc.subcore_barrier()
    ebase = (c * NS + s) * EPT

    def body(i, _):
      base = ebase + i * CHUNK
      pltpu.sync_copy(src_hbm.at[pl.ds(base, CHUNK)], sidx)
      pltpu.sync_copy(dst_hbm.at[pl.ds(base, CHUNK)], didx)
      pltpu.async_copy(table_hbm.at[sidx], rows, sem).wait()
      pltpu.sync_copy(rows, acc.at[didx], add=True)
      return ()

    lax.fori_loop(0, NCHUNK, body, ())
    plsc.subcore_barrier()
    pltpu.sync_copy(acc.at[pl.ds(r0, ROWS_PT)],
                    out_hbm.at[c].at[pl.ds(r0, ROWS_PT)])

  return prop


_prop128 = _make_prop(128)
_prop32 = _make_prop(32)
_prop16 = _make_prop(16)


def _tc_k1_body(d0_ref, d1_ref, x_ref, w1_ref, dinv_ref, hn1_ref):
  deg = d0_ref[...] + d1_ref[...] + 1.0
  dinv = lax.rsqrt(deg)
  dinv_ref[...] = dinv
  h = jnp.dot(x_ref[...], w1_ref[...], preferred_element_type=jnp.float32)
  hn1_ref[...] = h * dinv


def _tc_k1(d0, d1, x, w1):
  grid = (N // ROW_BLK,)
  return pl.pallas_call(
      _tc_k1_body,
      grid=grid,
      in_specs=[
          pl.BlockSpec((ROW_BLK, 1), lambda i: (i, 0)),
          pl.BlockSpec((ROW_BLK, 1), lambda i: (i, 0)),
          pl.BlockSpec((ROW_BLK, 128), lambda i: (i, 0)),
          pl.BlockSpec((128, 128), lambda i: (0, 0)),
      ],
      out_specs=[
          pl.BlockSpec((ROW_BLK, 1), lambda i: (i, 0)),
          pl.BlockSpec((ROW_BLK, 128), lambda i: (i, 0)),
      ],
      out_shape=[
          jax.ShapeDtypeStruct((N, 1), jnp.float32),
          jax.ShapeDtypeStruct((N, 128), jnp.float32),
      ],
  )(d0, d1, x, w1)


def _make_tc_mid(d_in, d_out):
  def body(sa_ref, sb_ref, hn_ref, dinv_ref, b_ref, w_ref, out_ref):
    dinv = dinv_ref[...]
    t = dinv * (sa_ref[...] + sb_ref[...] + hn_ref[...]) + b_ref[...]
    t = jnp.maximum(t, 0.0)
    h = jnp.dot(t, w_ref[...], preferred_element_type=jnp.float32)
    out_ref[...] = h * dinv

  def run(sa, sb, hn, dinv, b, w):
    grid = (N // ROW_BLK,)
    return pl.pallas_call(
        body,
        grid=grid,
        in_specs=[
            pl.BlockSpec((ROW_BLK, d_in), lambda i: (i, 0)),
            pl.BlockSpec((ROW_BLK, d_in), lambda i: (i, 0)),
            pl.BlockSpec((ROW_BLK, d_in), lambda i: (i, 0)),
            pl.BlockSpec((ROW_BLK, 1), lambda i: (i, 0)),
            pl.BlockSpec((1, d_in), lambda i: (0, 0)),
            pl.BlockSpec((d_in, d_out), lambda i: (0, 0)),
        ],
        out_specs=pl.BlockSpec((ROW_BLK, d_out), lambda i: (i, 0)),
        out_shape=jax.ShapeDtypeStruct((N, d_out), jnp.float32),
    )(sa, sb, hn, dinv, b, w)

  return run


_tc_k2 = _make_tc_mid(128, 32)
_tc_k3 = _make_tc_mid(32, 16)


def _tc_k4_body(sa_ref, sb_ref, hn_ref, dinv_ref, b_ref, out_ref):
  o = dinv_ref[...] * (sa_ref[...] + sb_ref[...] + hn_ref[...]) + b_ref[...]
  m = jnp.max(o, axis=1, keepdims=True)
  e = jnp.exp(o - m)
  lse = m + jnp.log(jnp.sum(e, axis=1, keepdims=True))
  out_ref[...] = o - lse


def _tc_k4(sa, sb, hn, dinv, b):
  grid = (N // ROW_BLK,)
  return pl.pallas_call(
      _tc_k4_body,
      grid=grid,
      in_specs=[
          pl.BlockSpec((ROW_BLK, 16), lambda i: (i, 0)),
          pl.BlockSpec((ROW_BLK, 16), lambda i: (i, 0)),
          pl.BlockSpec((ROW_BLK, 16), lambda i: (i, 0)),
          pl.BlockSpec((ROW_BLK, 1), lambda i: (i, 0)),
          pl.BlockSpec((1, 16), lambda i: (0, 0)),
      ],
      out_specs=pl.BlockSpec((ROW_BLK, 16), lambda i: (i, 0)),
      out_shape=jax.ShapeDtypeStruct((N, 16), jnp.float32),
  )(sa, sb, hn, dinv, b)


def kernel(x, edge_index, W1, b1, W2, b2, W3, b3):
  src = edge_index[0]
  dst = edge_index[1]

  ones_c = jnp.ones((CHUNK, DEG_W), jnp.float32)
  zeros_deg = jnp.zeros((N_PAD, DEG_W), jnp.float32)
  degp = _sc_degree(dst, ones_c, zeros_deg)
  d0 = degp[0, :, 0:1]
  d1 = degp[1, :, 0:1]

  dinv, hn1 = _tc_k1(d0, d1, x, W1)

  z128 = jnp.zeros((N_PAD, 128), jnp.float32)
  s1 = _prop128(hn1, src, dst, z128)
  hn2 = _tc_k2(s1[0], s1[1], hn1, dinv, b1.reshape(1, 128), W2)

  z32 = jnp.zeros((N_PAD, 32), jnp.float32)
  s2 = _prop32(hn2, src, dst, z32)
  hn3 = _tc_k3(s2[0], s2[1], hn2, dinv, b2.reshape(1, 32), W3)

  z16 = jnp.zeros((N_PAD, 16), jnp.float32)
  s3 = _prop16(hn3, src, dst, z16)
  return _tc_k4(s3[0], s3[1], hn3, dinv, b3.reshape(1, 16))


# speedup vs baseline: 10.0128x; 1.0286x over previous
import functools

import jax
import jax.numpy as jnp
from jax import lax
from jax.experimental import pallas as pl
from jax.experimental.pallas import tpu as pltpu
from jax.experimental.pallas import tpu_sc as plsc

N = 10000
N_PAD = 10240
E = 320000
NC = 2
NS = 16
NW = NC * NS
CHUNK = 128
NCHUNK = 80
EPT = NCHUNK * CHUNK
E_PAD = NW * EPT
K = 2
PAIRS = NCHUNK // (2 * K)
ROWS_PT = N_PAD // NS
DEG_W = 16

ROW_BLK = 400


def _sc_mesh():
  return plsc.VectorSubcoreMesh(core_axis_name="c", subcore_axis_name="s")


_DK = 8


@functools.partial(
    pl.kernel,
    out_type=jax.ShapeDtypeStruct((NC, N_PAD, DEG_W), jnp.float32),
    mesh=_sc_mesh(),
    scratch_types=[
        pltpu.VMEM((NCHUNK, CHUNK), jnp.int32),
        pltpu.VMEM((CHUNK, DEG_W), jnp.float32),
        pltpu.VMEM_SHARED((N_PAD, DEG_W), jnp.float32),
        pltpu.SemaphoreType.DMA,
    ],
    compiler_params=pltpu.CompilerParams(use_tc_tiling_on_sc=False),
)
def _sc_degree(dst3_hbm, ones_hbm, zeros_hbm, out_hbm, didx2, ones_v, acc,
               ssem):
  c = lax.axis_index("c")
  s = lax.axis_index("s")
  r0 = s * ROWS_PT
  w = c * NS + s
  pltpu.sync_copy(dst3_hbm.at[w], didx2)
  pltpu.sync_copy(ones_hbm, ones_v)
  pltpu.sync_copy(zeros_hbm.at[pl.ds(r0, ROWS_PT)], acc.at[pl.ds(r0, ROWS_PT)])
  plsc.subcore_barrier()

  def body(g, _):
    for k in range(_DK):
      pltpu.async_copy(ones_v, acc.at[didx2.at[g * _DK + k]], ssem, add=True)
    for k in range(_DK):
      pltpu.make_async_copy(ones_v, acc.at[didx2.at[g * _DK + k]], ssem).wait()
    return ()

  lax.fori_loop(0, NCHUNK // _DK, body, ())
  plsc.subcore_barrier()
  pltpu.sync_copy(acc.at[pl.ds(r0, ROWS_PT)],
                  out_hbm.at[c].at[pl.ds(r0, ROWS_PT)])


IH = NCHUNK // 2


def _make_prop(D):
  @functools.partial(
      pl.kernel,
      out_type=jax.ShapeDtypeStruct((NC, N_PAD, D), jnp.float32),
      mesh=_sc_mesh(),
      scratch_types=[
          pltpu.VMEM((IH, CHUNK), jnp.int32),
          pltpu.VMEM((IH, CHUNK), jnp.int32),
          [pltpu.VMEM((CHUNK, D), jnp.float32) for _ in range(2)],
          pltpu.VMEM_SHARED((N_PAD, D), jnp.float32),
          [pltpu.SemaphoreType.DMA for _ in range(2)],
          [pltpu.SemaphoreType.DMA for _ in range(2)],
      ],
      compiler_params=pltpu.CompilerParams(use_tc_tiling_on_sc=False),
  )
  def prop(table_hbm, src3_hbm, dst3_hbm, zeros_hbm, out_hbm,
           sidx2, didx2, rows, acc, gsems, ssems):
    c = lax.axis_index("c")
    s = lax.axis_index("s")
    r0 = s * ROWS_PT
    w = c * NS + s
    pltpu.sync_copy(zeros_hbm.at[pl.ds(r0, ROWS_PT)],
                    acc.at[pl.ds(r0, ROWS_PT)])
    plsc.subcore_barrier()

    def fire_gather(b, i):
      pltpu.async_copy(table_hbm.at[sidx2.at[i]], rows[b], gsems[b])

    def wait_gather(b, i):
      pltpu.make_async_copy(table_hbm.at[sidx2.at[i]], rows[b],
                            gsems[b]).wait()

    def fire_scatter(b, i):
      pltpu.async_copy(rows[b], acc.at[didx2.at[i]], ssems[b], add=True)

    def wait_scatter(b, i):
      pltpu.make_async_copy(rows[b], acc.at[didx2.at[i]], ssems[b]).wait()

    for phase in range(2):
      base = phase * IH
      pltpu.sync_copy(src3_hbm.at[w].at[pl.ds(base, IH)], sidx2)
      pltpu.sync_copy(dst3_hbm.at[w].at[pl.ds(base, IH)], didx2)
      fire_gather(0, 0)
      wait_gather(0, 0)
      fire_scatter(0, 0)
      fire_gather(1, 1)

      def body(j, _):
        i1 = 2 * j + 1
        wait_gather(1, i1)
        wait_scatter(0, i1 - 1)
        fire_scatter(1, i1)
        fire_gather(0, i1 + 1)
        i2 = i1 + 1
        wait_gather(0, i2)
        wait_scatter(1, i2 - 1)
        fire_scatter(0, i2)
        fire_gather(1, i2 + 1)
        return ()

      lax.fori_loop(0, IH // 2 - 1, body, ())
      last = IH - 1
      wait_gather(1, last)
      wait_scatter(0, last - 1)
      fire_scatter(1, last)
      wait_scatter(1, last)

    plsc.subcore_barrier()
    pltpu.sync_copy(acc.at[pl.ds(r0, ROWS_PT)],
                    out_hbm.at[c].at[pl.ds(r0, ROWS_PT)])

  return prop


_prop128 = _make_prop(128)
_prop32 = _make_prop(32)
_prop16 = _make_prop(16)


def _tc_k1_body(d0_ref, d1_ref, x_ref, w1_ref, dinv_ref, hn1_ref):
  deg = d0_ref[...] + d1_ref[...] + 1.0
  dinv = lax.rsqrt(deg)
  dinv_ref[...] = dinv
  h = jnp.dot(x_ref[...], w1_ref[...], preferred_element_type=jnp.float32)
  hn1_ref[...] = h * dinv


def _tc_k1(d0, d1, x, w1):
  grid = (N // ROW_BLK,)
  return pl.pallas_call(
      _tc_k1_body,
      grid=grid,
      in_specs=[
          pl.BlockSpec((ROW_BLK, 1), lambda i: (i, 0)),
          pl.BlockSpec((ROW_BLK, 1), lambda i: (i, 0)),
          pl.BlockSpec((ROW_BLK, 128), lambda i: (i, 0)),
          pl.BlockSpec((128, 128), lambda i: (0, 0)),
      ],
      out_specs=[
          pl.BlockSpec((ROW_BLK, 1), lambda i: (i, 0)),
          pl.BlockSpec((ROW_BLK, 128), lambda i: (i, 0)),
      ],
      out_shape=[
          jax.ShapeDtypeStruct((N, 1), jnp.float32),
          jax.ShapeDtypeStruct((N, 128), jnp.float32),
      ],
  )(d0, d1, x, w1)


def _make_tc_mid(d_in, d_out):
  def body(sa_ref, sb_ref, hn_ref, dinv_ref, b_ref, w_ref, out_ref):
    dinv = dinv_ref[...]
    t = dinv * (sa_ref[...] + sb_ref[...] + hn_ref[...]) + b_ref[...]
    t = jnp.maximum(t, 0.0)
    h = jnp.dot(t, w_ref[...], preferred_element_type=jnp.float32)
    out_ref[...] = h * dinv

  def run(sa, sb, hn, dinv, b, w):
    grid = (N // ROW_BLK,)
    return pl.pallas_call(
        body,
        grid=grid,
        in_specs=[
            pl.BlockSpec((ROW_BLK, d_in), lambda i: (i, 0)),
            pl.BlockSpec((ROW_BLK, d_in), lambda i: (i, 0)),
            pl.BlockSpec((ROW_BLK, d_in), lambda i: (i, 0)),
            pl.BlockSpec((ROW_BLK, 1), lambda i: (i, 0)),
            pl.BlockSpec((1, d_in), lambda i: (0, 0)),
            pl.BlockSpec((d_in, d_out), lambda i: (0, 0)),
        ],
        out_specs=pl.BlockSpec((ROW_BLK, d_out), lambda i: (i, 0)),
        out_shape=jax.ShapeDtypeStruct((N, d_out), jnp.float32),
    )(sa, sb, hn, dinv, b, w)

  return run


_tc_k2 = _make_tc_mid(128, 32)
_tc_k3 = _make_tc_mid(32, 16)


def _tc_k4_body(sa_ref, sb_ref, hn_ref, dinv_ref, b_ref, out_ref):
  o = dinv_ref[...] * (sa_ref[...] + sb_ref[...] + hn_ref[...]) + b_ref[...]
  m = jnp.max(o, axis=1, keepdims=True)
  e = jnp.exp(o - m)
  lse = m + jnp.log(jnp.sum(e, axis=1, keepdims=True))
  out_ref[...] = o - lse


def _tc_k4(sa, sb, hn, dinv, b):
  grid = (N // ROW_BLK,)
  return pl.pallas_call(
      _tc_k4_body,
      grid=grid,
      in_specs=[
          pl.BlockSpec((ROW_BLK, 16), lambda i: (i, 0)),
          pl.BlockSpec((ROW_BLK, 16), lambda i: (i, 0)),
          pl.BlockSpec((ROW_BLK, 16), lambda i: (i, 0)),
          pl.BlockSpec((ROW_BLK, 1), lambda i: (i, 0)),
          pl.BlockSpec((1, 16), lambda i: (0, 0)),
      ],
      out_specs=pl.BlockSpec((ROW_BLK, 16), lambda i: (i, 0)),
      out_shape=jax.ShapeDtypeStruct((N, 16), jnp.float32),
  )(sa, sb, hn, dinv, b)


def kernel(x, edge_index, W1, b1, W2, b2, W3, b3):
  src = edge_index[0]
  dst = edge_index[1]

  pad_e = E_PAD - E
  pad_dst = N + (jnp.arange(pad_e, dtype=jnp.int32) % (N_PAD - N))
  src3 = jnp.concatenate([src, jnp.zeros((pad_e,), jnp.int32)]
                         ).reshape(NW, NCHUNK, CHUNK)
  dst3 = jnp.concatenate([dst, pad_dst]).reshape(NW, NCHUNK, CHUNK)

  ones_c = jnp.ones((CHUNK, DEG_W), jnp.float32)
  zeros_deg = jnp.zeros((N_PAD, DEG_W), jnp.float32)
  degp = _sc_degree(dst3, ones_c, zeros_deg)
  d0 = degp[0, :, 0:1]
  d1 = degp[1, :, 0:1]

  dinv, hn1 = _tc_k1(d0, d1, x, W1)

  z128 = jnp.zeros((N_PAD, 128), jnp.float32)
  s1 = _prop128(hn1, src3, dst3, z128)
  hn2 = _tc_k2(s1[0], s1[1], hn1, dinv, b1.reshape(1, 128), W2)

  z32 = jnp.zeros((N_PAD, 32), jnp.float32)
  s2 = _prop32(hn2, src3, dst3, z32)
  hn3 = _tc_k3(s2[0], s2[1], hn2, dinv, b2.reshape(1, 32), W3)

  z16 = jnp.zeros((N_PAD, 16), jnp.float32)
  s3 = _prop16(hn3, src3, dst3, z16)
  return _tc_k4(s3[0], s3[1], hn3, dinv, b3.reshape(1, 16))


# speedup vs baseline: 11.2344x; 1.1220x over previous
import functools

import jax
import jax.numpy as jnp
from jax import lax
from jax.experimental import pallas as pl
from jax.experimental.pallas import tpu as pltpu
from jax.experimental.pallas import tpu_sc as plsc

N = 10000
N_PAD = 10240
E = 320000
NC = 2
NS = 16
NW = NC * NS
CHUNK = 128
NCHUNK = 80
EPT = NCHUNK * CHUNK
E_PAD = NW * EPT
K = 2
PAIRS = NCHUNK // (2 * K)
ROWS_PT = N_PAD // NS
DEG_W = 16

ROW_BLK = 400


def _sc_mesh():
  return plsc.VectorSubcoreMesh(core_axis_name="c", subcore_axis_name="s")


_DK = 8


@functools.partial(
    pl.kernel,
    out_type=jax.ShapeDtypeStruct((NC, N_PAD, DEG_W), jnp.float32),
    mesh=_sc_mesh(),
    scratch_types=[
        pltpu.VMEM((NCHUNK, CHUNK), jnp.int32),
        pltpu.VMEM((CHUNK, DEG_W), jnp.float32),
        pltpu.VMEM_SHARED((N_PAD, DEG_W), jnp.float32),
        pltpu.SemaphoreType.DMA,
    ],
    compiler_params=pltpu.CompilerParams(use_tc_tiling_on_sc=False),
)
def _sc_degree(dst3_hbm, ones_hbm, zeros_hbm, out_hbm, didx2, ones_v, acc,
               ssem):
  c = lax.axis_index("c")
  s = lax.axis_index("s")
  r0 = s * ROWS_PT
  w = c * NS + s
  pltpu.sync_copy(dst3_hbm.at[w], didx2)
  pltpu.sync_copy(ones_hbm, ones_v)
  pltpu.sync_copy(zeros_hbm.at[pl.ds(r0, ROWS_PT)], acc.at[pl.ds(r0, ROWS_PT)])
  plsc.subcore_barrier()

  def body(g, _):
    for k in range(_DK):
      pltpu.async_copy(ones_v, acc.at[didx2.at[g * _DK + k]], ssem, add=True)
    for k in range(_DK):
      pltpu.make_async_copy(ones_v, acc.at[didx2.at[g * _DK + k]], ssem).wait()
    return ()

  lax.fori_loop(0, NCHUNK // _DK, body, ())
  plsc.subcore_barrier()
  pltpu.sync_copy(acc.at[pl.ds(r0, ROWS_PT)],
                  out_hbm.at[c].at[pl.ds(r0, ROWS_PT)])


IH = NCHUNK // 2


def _make_prop(D):
  @functools.partial(
      pl.kernel,
      out_type=jax.ShapeDtypeStruct((NC, N_PAD, D), jnp.float32),
      mesh=_sc_mesh(),
      scratch_types=[
          pltpu.VMEM((IH, CHUNK), jnp.int32),
          pltpu.VMEM((IH, CHUNK), jnp.int32),
          [pltpu.VMEM((CHUNK, D), jnp.float32) for _ in range(2)],
          pltpu.VMEM_SHARED((N_PAD, D), jnp.float32),
          [pltpu.SemaphoreType.DMA for _ in range(2)],
          [pltpu.SemaphoreType.DMA for _ in range(2)],
      ],
      compiler_params=pltpu.CompilerParams(use_tc_tiling_on_sc=False),
  )
  def prop(table_hbm, src3_hbm, dst3_hbm, zeros_hbm, out_hbm,
           sidx2, didx2, rows, acc, gsems, ssems):
    c = lax.axis_index("c")
    s = lax.axis_index("s")
    r0 = s * ROWS_PT
    w = c * NS + s
    pltpu.sync_copy(zeros_hbm.at[pl.ds(r0, ROWS_PT)],
                    acc.at[pl.ds(r0, ROWS_PT)])
    plsc.subcore_barrier()

    def fire_gather(b, i):
      pltpu.async_copy(table_hbm.at[sidx2.at[i]], rows[b], gsems[b])

    def wait_gather(b, i):
      pltpu.make_async_copy(table_hbm.at[sidx2.at[i]], rows[b],
                            gsems[b]).wait()

    def fire_scatter(b, i):
      pltpu.async_copy(rows[b], acc.at[didx2.at[i]], ssems[b], add=True)

    def wait_scatter(b, i):
      pltpu.make_async_copy(rows[b], acc.at[didx2.at[i]], ssems[b]).wait()

    for phase in range(2):
      base = phase * IH
      pltpu.sync_copy(src3_hbm.at[w].at[pl.ds(base, IH)], sidx2)
      pltpu.sync_copy(dst3_hbm.at[w].at[pl.ds(base, IH)], didx2)
      fire_gather(0, 0)
      wait_gather(0, 0)
      fire_scatter(0, 0)
      fire_gather(1, 1)

      def body(j, _):
        i1 = 2 * j + 1
        wait_gather(1, i1)
        wait_scatter(0, i1 - 1)
        fire_scatter(1, i1)
        fire_gather(0, i1 + 1)
        i2 = i1 + 1
        wait_gather(0, i2)
        wait_scatter(1, i2 - 1)
        fire_scatter(0, i2)
        fire_gather(1, i2 + 1)
        return ()

      lax.fori_loop(0, IH // 2 - 1, body, ())
      last = IH - 1
      wait_gather(1, last)
      wait_scatter(0, last - 1)
      fire_scatter(1, last)
      wait_scatter(1, last)

    plsc.subcore_barrier()
    pltpu.sync_copy(acc.at[pl.ds(r0, ROWS_PT)],
                    out_hbm.at[c].at[pl.ds(r0, ROWS_PT)])

  return prop


_prop128 = _make_prop(128)
_prop32 = _make_prop(32)
_prop16 = _make_prop(16)


def _tc_k1_body(d0_ref, d1_ref, x_ref, w1_ref, dinv_ref, hn1_ref):
  deg = d0_ref[...] + d1_ref[...] + 1.0
  dinv = lax.rsqrt(deg)
  dinv_ref[...] = dinv
  h = jnp.dot(x_ref[...], w1_ref[...], preferred_element_type=jnp.float32)
  hn1_ref[...] = h * dinv


def _tc_k1(d0, d1, x, w1):
  grid = (N // ROW_BLK,)
  return pl.pallas_call(
      _tc_k1_body,
      grid=grid,
      in_specs=[
          pl.BlockSpec((ROW_BLK, 1), lambda i: (i, 0)),
          pl.BlockSpec((ROW_BLK, 1), lambda i: (i, 0)),
          pl.BlockSpec((ROW_BLK, 128), lambda i: (i, 0)),
          pl.BlockSpec((128, 128), lambda i: (0, 0)),
      ],
      out_specs=[
          pl.BlockSpec((ROW_BLK, 1), lambda i: (i, 0)),
          pl.BlockSpec((ROW_BLK, 128), lambda i: (i, 0)),
      ],
      out_shape=[
          jax.ShapeDtypeStruct((N, 1), jnp.float32),
          jax.ShapeDtypeStruct((N, 128), jnp.float32),
      ],
  )(d0, d1, x, w1)


def _make_tc_mid(d_in, d_out):
  def body(sa_ref, sb_ref, hn_ref, dinv_ref, b_ref, w_ref, out_ref):
    dinv = dinv_ref[...]
    t = dinv * (sa_ref[...] + sb_ref[...] + hn_ref[...]) + b_ref[...]
    t = jnp.maximum(t, 0.0)
    h = jnp.dot(t, w_ref[...], preferred_element_type=jnp.float32)
    out_ref[...] = h * dinv

  def run(sa, sb, hn, dinv, b, w):
    grid = (N // ROW_BLK,)
    return pl.pallas_call(
        body,
        grid=grid,
        in_specs=[
            pl.BlockSpec((ROW_BLK, d_in), lambda i: (i, 0)),
            pl.BlockSpec((ROW_BLK, d_in), lambda i: (i, 0)),
            pl.BlockSpec((ROW_BLK, d_in), lambda i: (i, 0)),
            pl.BlockSpec((ROW_BLK, 1), lambda i: (i, 0)),
            pl.BlockSpec((1, d_in), lambda i: (0, 0)),
            pl.BlockSpec((d_in, d_out), lambda i: (0, 0)),
        ],
        out_specs=pl.BlockSpec((ROW_BLK, d_out), lambda i: (i, 0)),
        out_shape=jax.ShapeDtypeStruct((N, d_out), jnp.float32),
    )(sa, sb, hn, dinv, b, w)

  return run


_tc_k2 = _make_tc_mid(128, 32)
_tc_k3 = _make_tc_mid(32, 16)


def _tc_k4_body(sa_ref, sb_ref, hn_ref, dinv_ref, b_ref, out_ref):
  o = dinv_ref[...] * (sa_ref[...] + sb_ref[...] + hn_ref[...]) + b_ref[...]
  m = jnp.max(o, axis=1, keepdims=True)
  e = jnp.exp(o - m)
  lse = m + jnp.log(jnp.sum(e, axis=1, keepdims=True))
  out_ref[...] = o - lse


def _tc_k4(sa, sb, hn, dinv, b):
  grid = (N // ROW_BLK,)
  return pl.pallas_call(
      _tc_k4_body,
      grid=grid,
      in_specs=[
          pl.BlockSpec((ROW_BLK, 16), lambda i: (i, 0)),
          pl.BlockSpec((ROW_BLK, 16), lambda i: (i, 0)),
          pl.BlockSpec((ROW_BLK, 16), lambda i: (i, 0)),
          pl.BlockSpec((ROW_BLK, 1), lambda i: (i, 0)),
          pl.BlockSpec((1, 16), lambda i: (0, 0)),
      ],
      out_specs=pl.BlockSpec((ROW_BLK, 16), lambda i: (i, 0)),
      out_shape=jax.ShapeDtypeStruct((N, 16), jnp.float32),
  )(sa, sb, hn, dinv, b)


def kernel(x, edge_index, W1, b1, W2, b2, W3, b3):
  src = edge_index[0]
  dst = edge_index[1]

  real_pt = E // NW
  pad_pt = EPT - real_pt
  pad_dst = jnp.broadcast_to(N + jnp.arange(pad_pt, dtype=jnp.int32),
                             (NW, pad_pt))
  src3 = jnp.concatenate(
      [src.reshape(NW, real_pt), jnp.zeros((NW, pad_pt), jnp.int32)], axis=1
  ).reshape(NW, NCHUNK, CHUNK)
  dst3 = jnp.concatenate(
      [dst.reshape(NW, real_pt), pad_dst], axis=1
  ).reshape(NW, NCHUNK, CHUNK)

  ones_c = jnp.ones((CHUNK, DEG_W), jnp.float32)
  zeros_deg = jnp.zeros((N_PAD, DEG_W), jnp.float32)
  degp = _sc_degree(dst3, ones_c, zeros_deg)
  d0 = degp[0, :, 0:1]
  d1 = degp[1, :, 0:1]

  dinv, hn1 = _tc_k1(d0, d1, x, W1)

  z128 = jnp.zeros((N_PAD, 128), jnp.float32)
  s1 = _prop128(hn1, src3, dst3, z128)
  hn2 = _tc_k2(s1[0], s1[1], hn1, dinv, b1.reshape(1, 128), W2)

  z32 = jnp.zeros((N_PAD, 32), jnp.float32)
  s2 = _prop32(hn2, src3, dst3, z32)
  hn3 = _tc_k3(s2[0], s2[1], hn2, dinv, b2.reshape(1, 32), W3)

  z16 = jnp.zeros((N_PAD, 16), jnp.float32)
  s3 = _prop16(hn3, src3, dst3, z16)
  return _tc_k4(s3[0], s3[1], hn3, dinv, b3.reshape(1, 16))


# speedup vs baseline: 20.5840x; 1.8322x over previous
import functools

import jax
import jax.numpy as jnp
from jax import lax
from jax.experimental import pallas as pl
from jax.experimental.pallas import tpu as pltpu
from jax.experimental.pallas import tpu_sc as plsc

N = 10000
N_PAD = 10240
E = 320000
NC = 2
NS = 16
NW = NC * NS
CHUNK = 128
NCHUNK = 80
EPT = NCHUNK * CHUNK
E_PAD = NW * EPT
K = 2
PAIRS = NCHUNK // (2 * K)
ROWS_PT = N_PAD // NS
DEG_W = 16

ROW_BLK = 512
ROW_BLK4 = 400


def _sc_mesh():
  return plsc.VectorSubcoreMesh(core_axis_name="c", subcore_axis_name="s")


_DK = 8


@functools.partial(
    pl.kernel,
    out_type=jax.ShapeDtypeStruct((NC, N_PAD, DEG_W), jnp.float32),
    mesh=_sc_mesh(),
    scratch_types=[
        pltpu.VMEM((NCHUNK, CHUNK), jnp.int32),
        pltpu.VMEM((CHUNK, DEG_W), jnp.float32),
        pltpu.VMEM_SHARED((N_PAD, DEG_W), jnp.float32),
        pltpu.SemaphoreType.DMA,
    ],
    compiler_params=pltpu.CompilerParams(use_tc_tiling_on_sc=False),
)
def _sc_degree(dst3_hbm, ones_hbm, zeros_hbm, out_hbm, didx2, ones_v, acc,
               ssem):
  c = lax.axis_index("c")
  s = lax.axis_index("s")
  r0 = s * ROWS_PT
  w = c * NS + s
  pltpu.sync_copy(dst3_hbm.at[w], didx2)
  pltpu.sync_copy(ones_hbm, ones_v)
  pltpu.sync_copy(zeros_hbm.at[pl.ds(r0, ROWS_PT)], acc.at[pl.ds(r0, ROWS_PT)])
  plsc.subcore_barrier()

  def body(g, _):
    for k in range(_DK):
      pltpu.async_copy(ones_v, acc.at[didx2.at[g * _DK + k]], ssem, add=True)
    for k in range(_DK):
      pltpu.make_async_copy(ones_v, acc.at[didx2.at[g * _DK + k]], ssem).wait()
    return ()

  lax.fori_loop(0, NCHUNK // _DK, body, ())
  plsc.subcore_barrier()
  pltpu.sync_copy(acc.at[pl.ds(r0, ROWS_PT)],
                  out_hbm.at[c].at[pl.ds(r0, ROWS_PT)])


IH = NCHUNK // 2


def _make_prop(D):
  @functools.partial(
      pl.kernel,
      out_type=jax.ShapeDtypeStruct((NC, N_PAD, D), jnp.float32),
      mesh=_sc_mesh(),
      scratch_types=[
          pltpu.VMEM((IH, CHUNK), jnp.int32),
          pltpu.VMEM((IH, CHUNK), jnp.int32),
          [pltpu.VMEM((CHUNK, D), jnp.float32) for _ in range(2)],
          pltpu.VMEM_SHARED((N_PAD, D), jnp.float32),
          [pltpu.SemaphoreType.DMA for _ in range(2)],
          [pltpu.SemaphoreType.DMA for _ in range(2)],
      ],
      compiler_params=pltpu.CompilerParams(use_tc_tiling_on_sc=False),
  )
  def prop(table_hbm, src3_hbm, dst3_hbm, zeros_hbm, out_hbm,
           sidx2, didx2, rows, acc, gsems, ssems):
    c = lax.axis_index("c")
    s = lax.axis_index("s")
    r0 = s * ROWS_PT
    w = c * NS + s
    pltpu.sync_copy(zeros_hbm.at[pl.ds(r0, ROWS_PT)],
                    acc.at[pl.ds(r0, ROWS_PT)])
    plsc.subcore_barrier()

    def fire_gather(b, i):
      pltpu.async_copy(table_hbm.at[sidx2.at[i]], rows[b], gsems[b])

    def wait_gather(b, i):
      pltpu.make_async_copy(table_hbm.at[sidx2.at[i]], rows[b],
                            gsems[b]).wait()

    def fire_scatter(b, i):
      pltpu.async_copy(rows[b], acc.at[didx2.at[i]], ssems[b], add=True)

    def wait_scatter(b, i):
      pltpu.make_async_copy(rows[b], acc.at[didx2.at[i]], ssems[b]).wait()

    for phase in range(2):
      base = phase * IH
      pltpu.sync_copy(src3_hbm.at[w].at[pl.ds(base, IH)], sidx2)
      pltpu.sync_copy(dst3_hbm.at[w].at[pl.ds(base, IH)], didx2)
      fire_gather(0, 0)
      wait_gather(0, 0)
      fire_scatter(0, 0)
      fire_gather(1, 1)

      def body(j, _):
        i1 = 2 * j + 1
        wait_gather(1, i1)
        wait_scatter(0, i1 - 1)
        fire_scatter(1, i1)
        fire_gather(0, i1 + 1)
        i2 = i1 + 1
        wait_gather(0, i2)
        wait_scatter(1, i2 - 1)
        fire_scatter(0, i2)
        fire_gather(1, i2 + 1)
        return ()

      lax.fori_loop(0, IH // 2 - 1, body, ())
      last = IH - 1
      wait_gather(1, last)
      wait_scatter(0, last - 1)
      fire_scatter(1, last)
      wait_scatter(1, last)

    plsc.subcore_barrier()
    pltpu.sync_copy(acc.at[pl.ds(r0, ROWS_PT)],
                    out_hbm.at[c].at[pl.ds(r0, ROWS_PT)])

  return prop


_prop128 = _make_prop(128)
_prop32 = _make_prop(32)
_prop16 = _make_prop(16)


def _row_valid():
  rid = pl.program_id(0) * ROW_BLK + lax.broadcasted_iota(
      jnp.int32, (ROW_BLK, 1), 0)
  return rid < N


def _tc_k1_body(d0_ref, d1_ref, x_ref, w1_ref, dinv_ref, hn1_ref):
  deg = d0_ref[...] + d1_ref[...] + 1.0
  dinv = lax.rsqrt(deg)
  dinv_ref[...] = dinv
  h = jnp.dot(x_ref[...], w1_ref[...], preferred_element_type=jnp.float32)
  hn1_ref[...] = jnp.where(_row_valid(), h * dinv, 0.0)


def _tc_k1(d0, d1, x, w1):
  grid = (N_PAD // ROW_BLK,)
  return pl.pallas_call(
      _tc_k1_body,
      grid=grid,
      in_specs=[
          pl.BlockSpec((ROW_BLK, 1), lambda i: (i, 0)),
          pl.BlockSpec((ROW_BLK, 1), lambda i: (i, 0)),
          pl.BlockSpec((ROW_BLK, 128), lambda i: (i, 0)),
          pl.BlockSpec((128, 128), lambda i: (0, 0)),
      ],
      out_specs=[
          pl.BlockSpec((ROW_BLK, 1), lambda i: (i, 0)),
          pl.BlockSpec((ROW_BLK, 128), lambda i: (i, 0)),
      ],
      out_shape=[
          jax.ShapeDtypeStruct((N_PAD, 1), jnp.float32),
          jax.ShapeDtypeStruct((N_PAD, 128), jnp.float32),
      ],
  )(d0, d1, x, w1)


def _make_tc_mid(d_in, d_out):
  def body(sa_ref, sb_ref, hn_ref, dinv_ref, b_ref, w_ref, out_ref):
    dinv = dinv_ref[...]
    t = dinv * (sa_ref[...] + sb_ref[...] + hn_ref[...]) + b_ref[...]
    t = jnp.maximum(t, 0.0)
    h = jnp.dot(t, w_ref[...], preferred_element_type=jnp.float32)
    out_ref[...] = jnp.where(_row_valid(), h * dinv, 0.0)

  def run(sa, sb, hn, dinv, b, w):
    grid = (N_PAD // ROW_BLK,)
    return pl.pallas_call(
        body,
        grid=grid,
        in_specs=[
            pl.BlockSpec((ROW_BLK, d_in), lambda i: (i, 0)),
            pl.BlockSpec((ROW_BLK, d_in), lambda i: (i, 0)),
            pl.BlockSpec((ROW_BLK, d_in), lambda i: (i, 0)),
            pl.BlockSpec((ROW_BLK, 1), lambda i: (i, 0)),
            pl.BlockSpec((1, d_in), lambda i: (0, 0)),
            pl.BlockSpec((d_in, d_out), lambda i: (0, 0)),
        ],
        out_specs=pl.BlockSpec((ROW_BLK, d_out), lambda i: (i, 0)),
        out_shape=jax.ShapeDtypeStruct((N_PAD, d_out), jnp.float32),
    )(sa, sb, hn, dinv, b, w)

  return run


_tc_k2 = _make_tc_mid(128, 32)
_tc_k3 = _make_tc_mid(32, 16)


def _tc_k4_body(sa_ref, sb_ref, hn_ref, dinv_ref, b_ref, out_ref):
  o = dinv_ref[...] * (sa_ref[...] + sb_ref[...] + hn_ref[...]) + b_ref[...]
  m = jnp.max(o, axis=1, keepdims=True)
  e = jnp.exp(o - m)
  lse = m + jnp.log(jnp.sum(e, axis=1, keepdims=True))
  out_ref[...] = o - lse


def _tc_k4(sa, sb, hn, dinv, b):
  grid = (N // ROW_BLK4,)
  return pl.pallas_call(
      _tc_k4_body,
      grid=grid,
      in_specs=[
          pl.BlockSpec((ROW_BLK4, 16), lambda i: (i, 0)),
          pl.BlockSpec((ROW_BLK4, 16), lambda i: (i, 0)),
          pl.BlockSpec((ROW_BLK4, 16), lambda i: (i, 0)),
          pl.BlockSpec((ROW_BLK4, 1), lambda i: (i, 0)),
          pl.BlockSpec((1, 16), lambda i: (0, 0)),
      ],
      out_specs=pl.BlockSpec((ROW_BLK4, 16), lambda i: (i, 0)),
      out_shape=jax.ShapeDtypeStruct((N, 16), jnp.float32),
  )(sa, sb, hn, dinv, b)


def kernel(x, edge_index, W1, b1, W2, b2, W3, b3):
  src = edge_index[0]
  dst = edge_index[1]

  real_pt = E // NW
  pad_pt = EPT - real_pt
  j = jnp.arange(pad_pt, dtype=jnp.int32)
  w = jnp.arange(NW, dtype=jnp.int32)[:, None]
  pad_dst_deg = jnp.broadcast_to(N + j, (NW, pad_pt))
  pad_src_prop = jnp.broadcast_to(N + j % (N_PAD - N), (NW, pad_pt))
  pad_dst_prop = ((w * pad_pt + j) * 37) % N_PAD
  src3 = jnp.concatenate(
      [src.reshape(NW, real_pt), pad_src_prop], axis=1
  ).reshape(NW, NCHUNK, CHUNK)
  dst3 = jnp.concatenate(
      [dst.reshape(NW, real_pt), pad_dst_prop], axis=1
  ).reshape(NW, NCHUNK, CHUNK)
  dst3_deg = jnp.concatenate(
      [dst.reshape(NW, real_pt), pad_dst_deg], axis=1
  ).reshape(NW, NCHUNK, CHUNK)

  ones_c = jnp.ones((CHUNK, DEG_W), jnp.float32)
  zeros_deg = jnp.zeros((N_PAD, DEG_W), jnp.float32)
  degp = _sc_degree(dst3_deg, ones_c, zeros_deg)
  d0 = degp[0, :, 0:1]
  d1 = degp[1, :, 0:1]

  dinv, hn1 = _tc_k1(d0, d1, x, W1)

  z128 = jnp.zeros((N_PAD, 128), jnp.float32)
  s1 = _prop128(hn1, src3, dst3, z128)
  hn2 = _tc_k2(s1[0], s1[1], hn1, dinv, b1.reshape(1, 128), W2)

  z32 = jnp.zeros((N_PAD, 32), jnp.float32)
  s2 = _prop32(hn2, src3, dst3, z32)
  hn3 = _tc_k3(s2[0], s2[1], hn2, dinv, b2.reshape(1, 32), W3)

  z16 = jnp.zeros((N_PAD, 16), jnp.float32)
  s3 = _prop16(hn3, src3, dst3, z16)
  return _tc_k4(s3[0], s3[1], hn3, dinv, b3.reshape(1, 16))


# speedup vs baseline: 23.1144x; 1.1229x over previous
import functools

import jax
import jax.numpy as jnp
from jax import lax
from jax.experimental import pallas as pl
from jax.experimental.pallas import tpu as pltpu
from jax.experimental.pallas import tpu_sc as plsc

N = 10000
N_PAD = 10240
E = 320000
NC = 2
NS = 16
NW = NC * NS
CHUNK = 128
NCHUNK = 80
EPT = NCHUNK * CHUNK
E_PAD = NW * EPT
K = 2
PAIRS = NCHUNK // (2 * K)
ROWS_PT = N_PAD // NS
DEG_W = 16

ROW_BLK = 512
ROW_BLK4 = 400


def _sc_mesh():
  return plsc.VectorSubcoreMesh(core_axis_name="c", subcore_axis_name="s")


_DK = 8


@functools.partial(
    pl.kernel,
    out_type=jax.ShapeDtypeStruct((NC, N_PAD, DEG_W), jnp.float32),
    mesh=_sc_mesh(),
    scratch_types=[
        pltpu.VMEM((NCHUNK, CHUNK), jnp.int32),
        pltpu.VMEM((CHUNK, DEG_W), jnp.float32),
        pltpu.VMEM_SHARED((N_PAD, DEG_W), jnp.float32),
        pltpu.SemaphoreType.DMA,
    ],
    compiler_params=pltpu.CompilerParams(use_tc_tiling_on_sc=False),
)
def _sc_degree(dst3_hbm, ones_hbm, zeros_hbm, out_hbm, didx2, ones_v, acc,
               ssem):
  c = lax.axis_index("c")
  s = lax.axis_index("s")
  r0 = s * ROWS_PT
  w = c * NS + s
  pltpu.sync_copy(dst3_hbm.at[w], didx2)
  pltpu.sync_copy(ones_hbm, ones_v)
  pltpu.sync_copy(zeros_hbm.at[pl.ds(r0, ROWS_PT)], acc.at[pl.ds(r0, ROWS_PT)])
  plsc.subcore_barrier()

  def body(g, _):
    for k in range(_DK):
      pltpu.async_copy(ones_v, acc.at[didx2.at[g * _DK + k]], ssem, add=True)
    for k in range(_DK):
      pltpu.make_async_copy(ones_v, acc.at[didx2.at[g * _DK + k]], ssem).wait()
    return ()

  lax.fori_loop(0, NCHUNK // _DK, body, ())
  plsc.subcore_barrier()
  pltpu.sync_copy(acc.at[pl.ds(r0, ROWS_PT)],
                  out_hbm.at[c].at[pl.ds(r0, ROWS_PT)])


IH = NCHUNK // 2


def _make_prop(D, K, G):
  S = K - G
  assert (IH - G - S) % K == 0

  @functools.partial(
      pl.kernel,
      out_type=jax.ShapeDtypeStruct((NC, N_PAD, D), jnp.float32),
      mesh=_sc_mesh(),
      scratch_types=[
          pltpu.VMEM((IH, CHUNK), jnp.int32),
          pltpu.VMEM((IH, CHUNK), jnp.int32),
          [pltpu.VMEM((CHUNK, D), jnp.float32) for _ in range(K)],
          pltpu.VMEM_SHARED((N_PAD, D), jnp.float32),
          [pltpu.SemaphoreType.DMA for _ in range(K)],
          [pltpu.SemaphoreType.DMA for _ in range(K)],
      ],
      compiler_params=pltpu.CompilerParams(use_tc_tiling_on_sc=False),
  )
  def prop(table_hbm, src3_hbm, dst3_hbm, zeros_hbm, out_hbm,
           sidx2, didx2, rows, acc, gsems, ssems):
    c = lax.axis_index("c")
    s = lax.axis_index("s")
    r0 = s * ROWS_PT
    w = c * NS + s
    pltpu.sync_copy(zeros_hbm.at[pl.ds(r0, ROWS_PT)],
                    acc.at[pl.ds(r0, ROWS_PT)])
    plsc.subcore_barrier()

    def fire_gather(b, i):
      pltpu.async_copy(table_hbm.at[sidx2.at[i]], rows[b], gsems[b])

    def wait_gather(b, i):
      pltpu.make_async_copy(table_hbm.at[sidx2.at[i]], rows[b],
                            gsems[b]).wait()

    def fire_scatter(b, i):
      pltpu.async_copy(rows[b], acc.at[didx2.at[i]], ssems[b], add=True)

    def wait_scatter(b, i):
      pltpu.make_async_copy(rows[b], acc.at[didx2.at[i]], ssems[b]).wait()

    for phase in range(2):
      base = phase * IH
      pltpu.sync_copy(src3_hbm.at[w].at[pl.ds(base, IH)], sidx2)
      pltpu.sync_copy(dst3_hbm.at[w].at[pl.ds(base, IH)], didx2)
      for i in range(G):
        fire_gather(i % K, i)
      for i in range(S):
        wait_gather(i % K, i)
        fire_scatter(i % K, i)
        fire_gather((i + G) % K, i + G)

      def body(j, _):
        for u in range(K):
          i = S + j * K + u
          b = (S + u) % K
          wait_gather(b, i)
          fire_scatter(b, i)
          wait_scatter(u % K, i - S)
          fire_gather((S + u + G) % K, i + G)
        return ()

      lax.fori_loop(0, (IH - G - S) // K, body, ())
      for i in range(IH - G, IH):
        wait_gather(i % K, i)
        fire_scatter(i % K, i)
        wait_scatter((i - S) % K, i - S)
      for i in range(IH - S, IH):
        wait_scatter(i % K, i)

    plsc.subcore_barrier()
    pltpu.sync_copy(acc.at[pl.ds(r0, ROWS_PT)],
                    out_hbm.at[c].at[pl.ds(r0, ROWS_PT)])

  return prop


_prop128 = _make_prop(128, 2, 1)
_prop32 = _make_prop(32, 4, 2)
_prop16 = _make_prop(16, 4, 2)


def _row_valid():
  rid = pl.program_id(0) * ROW_BLK + lax.broadcasted_iota(
      jnp.int32, (ROW_BLK, 1), 0)
  return rid < N


def _tc_k1a_body(x_ref, w1_ref, h_ref):
  h_ref[...] = jnp.dot(x_ref[...], w1_ref[...],
                       preferred_element_type=jnp.float32)


def _tc_k1a(x, w1):
  grid = (N_PAD // ROW_BLK,)
  return pl.pallas_call(
      _tc_k1a_body,
      grid=grid,
      in_specs=[
          pl.BlockSpec((ROW_BLK, 128), lambda i: (i, 0)),
          pl.BlockSpec((128, 128), lambda i: (0, 0)),
      ],
      out_specs=pl.BlockSpec((ROW_BLK, 128), lambda i: (i, 0)),
      out_shape=jax.ShapeDtypeStruct((N_PAD, 128), jnp.float32),
  )(x, w1)


def _tc_k1b_body(d0_ref, d1_ref, h_ref, dinv_ref, hn1_ref):
  deg = d0_ref[...] + d1_ref[...] + 1.0
  dinv = lax.rsqrt(deg)
  dinv_ref[...] = dinv
  hn1_ref[...] = jnp.where(_row_valid(), h_ref[...] * dinv, 0.0)


def _tc_k1b(d0, d1, h):
  grid = (N_PAD // ROW_BLK,)
  return pl.pallas_call(
      _tc_k1b_body,
      grid=grid,
      in_specs=[
          pl.BlockSpec((ROW_BLK, 1), lambda i: (i, 0)),
          pl.BlockSpec((ROW_BLK, 1), lambda i: (i, 0)),
          pl.BlockSpec((ROW_BLK, 128), lambda i: (i, 0)),
      ],
      out_specs=[
          pl.BlockSpec((ROW_BLK, 1), lambda i: (i, 0)),
          pl.BlockSpec((ROW_BLK, 128), lambda i: (i, 0)),
      ],
      out_shape=[
          jax.ShapeDtypeStruct((N_PAD, 1), jnp.float32),
          jax.ShapeDtypeStruct((N_PAD, 128), jnp.float32),
      ],
  )(d0, d1, h)


def _make_tc_mid(d_in, d_out):
  def body(sa_ref, sb_ref, hn_ref, dinv_ref, b_ref, w_ref, out_ref):
    dinv = dinv_ref[...]
    t = dinv * (sa_ref[...] + sb_ref[...] + hn_ref[...]) + b_ref[...]
    t = jnp.maximum(t, 0.0)
    h = jnp.dot(t, w_ref[...], preferred_element_type=jnp.float32)
    out_ref[...] = jnp.where(_row_valid(), h * dinv, 0.0)

  def run(sa, sb, hn, dinv, b, w):
    grid = (N_PAD // ROW_BLK,)
    return pl.pallas_call(
        body,
        grid=grid,
        in_specs=[
            pl.BlockSpec((ROW_BLK, d_in), lambda i: (i, 0)),
            pl.BlockSpec((ROW_BLK, d_in), lambda i: (i, 0)),
            pl.BlockSpec((ROW_BLK, d_in), lambda i: (i, 0)),
            pl.BlockSpec((ROW_BLK, 1), lambda i: (i, 0)),
            pl.BlockSpec((1, d_in), lambda i: (0, 0)),
            pl.BlockSpec((d_in, d_out), lambda i: (0, 0)),
        ],
        out_specs=pl.BlockSpec((ROW_BLK, d_out), lambda i: (i, 0)),
        out_shape=jax.ShapeDtypeStruct((N_PAD, d_out), jnp.float32),
    )(sa, sb, hn, dinv, b, w)

  return run


_tc_k2 = _make_tc_mid(128, 32)
_tc_k3 = _make_tc_mid(32, 16)


def _tc_k4_body(sa_ref, sb_ref, hn_ref, dinv_ref, b_ref, out_ref):
  o = dinv_ref[...] * (sa_ref[...] + sb_ref[...] + hn_ref[...]) + b_ref[...]
  m = jnp.max(o, axis=1, keepdims=True)
  e = jnp.exp(o - m)
  lse = m + jnp.log(jnp.sum(e, axis=1, keepdims=True))
  out_ref[...] = o - lse


def _tc_k4(sa, sb, hn, dinv, b):
  grid = (N // ROW_BLK4,)
  return pl.pallas_call(
      _tc_k4_body,
      grid=grid,
      in_specs=[
          pl.BlockSpec((ROW_BLK4, 16), lambda i: (i, 0)),
          pl.BlockSpec((ROW_BLK4, 16), lambda i: (i, 0)),
          pl.BlockSpec((ROW_BLK4, 16), lambda i: (i, 0)),
          pl.BlockSpec((ROW_BLK4, 1), lambda i: (i, 0)),
          pl.BlockSpec((1, 16), lambda i: (0, 0)),
      ],
      out_specs=pl.BlockSpec((ROW_BLK4, 16), lambda i: (i, 0)),
      out_shape=jax.ShapeDtypeStruct((N, 16), jnp.float32),
  )(sa, sb, hn, dinv, b)


def kernel(x, edge_index, W1, b1, W2, b2, W3, b3):
  src = edge_index[0]
  dst = edge_index[1]

  real_pt = E // NW
  pad_pt = EPT - real_pt
  j = jnp.arange(pad_pt, dtype=jnp.int32)
  w = jnp.arange(NW, dtype=jnp.int32)[:, None]
  pad_dst_deg = jnp.broadcast_to(N + j, (NW, pad_pt))
  pad_src_prop = jnp.broadcast_to(N + j % (N_PAD - N), (NW, pad_pt))
  pad_dst_prop = ((w * pad_pt + j) * 37) % N_PAD
  src3 = jnp.concatenate(
      [src.reshape(NW, real_pt), pad_src_prop], axis=1
  ).reshape(NW, NCHUNK, CHUNK)
  dst3 = jnp.concatenate(
      [dst.reshape(NW, real_pt), pad_dst_prop], axis=1
  ).reshape(NW, NCHUNK, CHUNK)
  dst3_deg = jnp.concatenate(
      [dst.reshape(NW, real_pt), pad_dst_deg], axis=1
  ).reshape(NW, NCHUNK, CHUNK)

  ones_c = jnp.ones((CHUNK, DEG_W), jnp.float32)
  zeros_deg = jnp.zeros((N_PAD, DEG_W), jnp.float32)
  degp = _sc_degree(dst3_deg, ones_c, zeros_deg)
  d0 = degp[0, :, 0:1]
  d1 = degp[1, :, 0:1]

  h1 = _tc_k1a(x, W1)
  dinv, hn1 = _tc_k1b(d0, d1, h1)

  z128 = jnp.zeros((N_PAD, 128), jnp.float32)
  s1 = _prop128(hn1, src3, dst3, z128)
  hn2 = _tc_k2(s1[0], s1[1], hn1, dinv, b1.reshape(1, 128), W2)

  z32 = jnp.zeros((N_PAD, 32), jnp.float32)
  s2 = _prop32(hn2, src3, dst3, z32)
  hn3 = _tc_k3(s2[0], s2[1], hn2, dinv, b2.reshape(1, 32), W3)

  z16 = jnp.zeros((N_PAD, 16), jnp.float32)
  s3 = _prop16(hn3, src3, dst3, z16)
  return _tc_k4(s3[0], s3[1], hn3, dinv, b3.reshape(1, 16))
